# baseline (speedup 1.0000x reference)
import functools

import jax
import jax.numpy as jnp
from jax import lax
from jax.experimental import pallas as pl
from jax.experimental.pallas import tpu as pltpu
from jax.experimental.pallas import tpu_sc as plsc

N_NODE = 10000
N_ATTRI = 2000
NN = N_NODE + N_ATTRI
NNP = 12032
E = 384000
D = 128
DH = D // 2
B = 16384
GB = 4 * B

NC = 2
NS = 16
NW = NC * NS

EPT = E // NS
EK = 96
SUBS = 5
NBODY = EPT // (EK * SUBS)
EBLK = E // EK

RPT = NNP // NS
GPT = GB // NW
GK = 128
NGCH = GPT // GK

_mesh = plsc.VectorSubcoreMesh(core_axis_name="c", subcore_axis_name="s")


@functools.partial(
    pl.kernel,
    mesh=_mesh,
    compiler_params=pltpu.CompilerParams(use_tc_tiling_on_sc=False),
    out_type=jax.ShapeDtypeStruct((GB, D), jnp.float32),
    scratch_types=[
        pltpu.VMEM((GPT,), jnp.int32),
        pltpu.VMEM((GK, D), jnp.float32),
        pltpu.VMEM((GK, D), jnp.float32),
        pltpu.VMEM((GK, D), jnp.float32),
        pltpu.VMEM((GK, D), jnp.float32),
        pltpu.SemaphoreType.DMA,
        pltpu.SemaphoreType.DMA,
        pltpu.SemaphoreType.DMA,
    ],
)
def _sc_gather(x_hbm, idx_hbm, out_hbm, idxv, A0, A1, O0, O1,
               gsem, osem, isem):
    wid = lax.axis_index("s") * NC + lax.axis_index("c")
    base = wid * GPT
    A = (A0, A1)
    O = (O0, O1)

    pltpu.async_copy(idx_hbm.at[pl.ds(base, GPT)], idxv, isem).wait()
    for k in range(2):
        pltpu.async_copy(x_hbm.at[idxv.at[pl.ds(k * GK, GK)]], A[k], gsem)

    def norm_chunk(a, o):
        def grp(it, carry):
            for u in range(4):
                r = it * 4 + u
                av = [a[r, pl.ds(cc * 16, 16)] for cc in range(D // 16)]
                sq = [x * x for x in av]
                while len(sq) > 1:
                    sq = [sq[2 * t] + sq[2 * t + 1] for t in range(len(sq) // 2)]
                bc = [jnp.full((16,), sq[0][e], jnp.float32) for e in range(16)]
                while len(bc) > 1:
                    bc = [bc[2 * t] + bc[2 * t + 1] for t in range(len(bc) // 2)]
                ss = jnp.maximum(bc[0], 1e-24)
                iy = jnp.int32(0x5F3759DF) - lax.shift_right_logical(
                    lax.bitcast_convert_type(ss, jnp.int32), 1)
                y = lax.bitcast_convert_type(iy, jnp.float32)
                y = y * (1.5 - 0.5 * ss * y * y)
                y = y * (1.5 - 0.5 * ss * y * y)
                y = y * (1.5 - 0.5 * ss * y * y)
                for cc in range(D // 16):
                    o[r, pl.ds(cc * 16, 16)] = av[cc] * y
            return carry

        lax.fori_loop(0, GK // 4, grp, 0)

    def step(i, k):
        j = 2 * i + k

        def drain_out():
            pltpu.make_async_copy(O[k], out_hbm.at[pl.ds(0, GK)], osem).wait()

        def fire_gather():
            pltpu.async_copy(x_hbm.at[idxv.at[pl.ds((j + 2) * GK, GK)]],
                             A[k], gsem)

        pltpu.make_async_copy(
            x_hbm.at[idxv.at[pl.ds(0, GK)]], A[k], gsem).wait()
        pl.when(i > 0)(drain_out)
        norm_chunk(A[k], O[k])
        pltpu.async_copy(O[k], out_hbm.at[pl.ds(base + j * GK, GK)], osem)
        pl.when(i < NGCH // 2 - 1)(fire_gather)

    def loop(i, carry):
        step(i, 0)
        step(i, 1)
        return carry

    lax.fori_loop(0, NGCH // 2, loop, 0)
    for k in range(2):
        pltpu.make_async_copy(O[k], out_hbm.at[pl.ds(0, GK)], osem).wait()


@functools.partial(
    pl.kernel,
    mesh=_mesh,
    compiler_params=pltpu.CompilerParams(use_tc_tiling_on_sc=False),
    out_type=jax.ShapeDtypeStruct((NNP, D), jnp.float32),
    scratch_types=[
        pltpu.VMEM((SUBS, EK), jnp.int32),
        pltpu.VMEM((SUBS, EK), jnp.int32),
        pltpu.VMEM((SUBS, EK), jnp.int32),
        pltpu.VMEM((SUBS, EK), jnp.int32),
        pltpu.VMEM((SUBS, EK), jnp.float32),
        pltpu.VMEM((SUBS, EK), jnp.float32),
        *[pltpu.VMEM((EK, DH // 2), jnp.int32) for _ in range(SUBS)],
        *[pltpu.VMEM((EK, DH), jnp.float32) for _ in range(SUBS)],
        *[pltpu.VMEM((EK,), jnp.int32) for _ in range(SUBS)],
        pltpu.VMEM((1, DH), jnp.float32),
        pltpu.VMEM_SHARED((NNP, DH), jnp.float32),
        pltpu.SemaphoreType.DMA,
        pltpu.SemaphoreType.DMA,
        pltpu.SemaphoreType.DMA,
        pltpu.SemaphoreType.DMA,
    ],
)
def _sc_spmm(sup_hbm, row_hbm, col_hbm, val_hbm, b_hbm, out_hbm,
             row0, row1, col0, col1, val0, val1,
             G0, G1, G2, G3, G4, S0, S1, S2, S3, S4, R0, R1, R2, R3, R4,
             bv, acc, gsem, ssem, isem0, isem1):
    G = (G0, G1, G2, G3, G4)
    S = (S0, S1, S2, S3, S4)
    R = (R0, R1, R2, R3, R4)
    core = lax.axis_index("c")
    s = lax.axis_index("s")
    rb0 = s * (EPT // EK)

    isems = (isem0, isem1)
    rows = (row0, row1)
    cols = (col0, col1)
    vals = (val0, val1)

    def fire_idx(p, rb):
        pltpu.async_copy(row_hbm.at[pl.ds(rb, SUBS)], rows[p], isems[p])
        pltpu.async_copy(col_hbm.at[pl.ds(rb, SUBS)], cols[p], isems[p])
        pltpu.async_copy(val_hbm.at[pl.ds(rb, SUBS)], vals[p], isems[p])

    def wait_idx(p):
        pltpu.make_async_copy(row_hbm.at[pl.ds(0, SUBS)], rows[p], isems[p]).wait()
        pltpu.make_async_copy(col_hbm.at[pl.ds(0, SUBS)], cols[p], isems[p]).wait()
        pltpu.make_async_copy(val_hbm.at[pl.ds(0, SUBS)], vals[p], isems[p]).wait()

    def xform_col_and_fire_gather(p, k):
        for g in range(EK // 16):
            sl = pl.ds(g * 16, 16)
            cols[p][k, sl] = cols[p][k, sl] * 2 + core
        pltpu.async_copy(sup_hbm.at[cols[p].at[k]], G[k], gsem)

    fire_idx(0, rb0)
    fire_idx(1, rb0 + SUBS)

    pltpu.sync_copy(b_hbm.at[:, pl.ds(core * DH, DH)], bv)
    bias = [bv[0, pl.ds(cc * 16, 16)] for cc in range(DH // 16)]

    def zrow(r, carry):
        for cc in range(DH // 16):
            S0[r, pl.ds(cc * 16, 16)] = bias[cc]
        return carry

    lax.fori_loop(0, EK, zrow, 0)
    _echunks = [(j * EK, EK) for j in range(RPT // EK)] + [
        ((RPT // EK) * EK, RPT - (RPT // EK) * EK)]
    for off, ln in _echunks:
        pltpu.sync_copy(S0.at[pl.ds(0, ln)], acc.at[pl.ds(s * RPT + off, ln)])
    plsc.subcore_barrier()

    wait_idx(0)
    for k in range(SUBS):
        xform_col_and_fire_gather(0, k)

    def scale(p, k):
        def sgrp(g, carry):
            vv = vals[p][k, pl.ds(g * 16, 16)]
            base = g * 16
            for e in range(16):
                v = jnp.full((16,), vv[e], jnp.float32)
                r = base + e
                tmp = []
                for g2 in range(DH // 32):
                    w = G[k][r, pl.ds(g2 * 16, 16)]
                    lo = lax.bitcast_convert_type(
                        lax.shift_left(w, 16), jnp.float32)
                    hi = lax.bitcast_convert_type(
                        lax.bitwise_and(w, jnp.int32(-65536)), jnp.float32)
                    tmp += [lo, hi]
                for cc in range(DH // 16):
                    S[k][r, pl.ds(cc * 16, 16)] = tmp[cc] * v
            return carry

        lax.fori_loop(0, EK // 16, sgrp, 0)

    def body(i, p):
        n = 1 - p
        for k in range(SUBS):
            def drain():
                pltpu.make_async_copy(S[k], acc.at[R[k]], ssem).wait()

            if p == 0:
                pl.when(i > 0)(drain)
            else:
                drain()
            pltpu.make_async_copy(sup_hbm.at[cols[p].at[k]], G[k], gsem).wait()
            if k == 0:
                if p == 0:
                    wait_idx(n)
                else:
                    pl.when(i < NBODY // 2 - 1)(lambda: wait_idx(n))
            for g in range(EK // 16):
                sl = pl.ds(g * 16, 16)
                R[k][sl] = rows[p][k, sl]
            scale(p, k)
            pltpu.async_copy(S[k], acc.at[R[k]], ssem, add=True)
            if p == 0:
                xform_col_and_fire_gather(n, k)
            else:
                pl.when(i < NBODY // 2 - 1)(
                    functools.partial(xform_col_and_fire_gather, n, k))
            if k == SUBS - 1:
                pl.when(i < NBODY // 2 - 1)(
                    lambda: fire_idx(p, rb0 + (2 * i + 2 + p) * SUBS))

    def loop(i, carry):
        body(i, 0)
        body(i, 1)
        return carry

    lax.fori_loop(0, NBODY // 2, loop, 0)

    for k in range(SUBS):
        pltpu.make_async_copy(S[k], acc.at[R[k]], ssem).wait()
    plsc.subcore_barrier()
    for j, (off, ln) in enumerate(_echunks):
        gb = S[j % 2]
        pltpu.sync_copy(acc.at[pl.ds(s * RPT + off, ln)], gb.at[pl.ds(0, ln)])
        pltpu.sync_copy(gb.at[pl.ds(0, ln)],
                        out_hbm.at[pl.ds(s * RPT + off, ln),
                                   pl.ds(core * DH, DH)])


_BM = 752


def _mm_body(x_ref, w_ref, o_ref):
    y = jnp.dot(x_ref[...], w_ref[...], preferred_element_type=jnp.float32)
    bm = y.shape[0]
    yp = y.reshape(bm, D // 32, 2, 16).transpose(0, 1, 3, 2).reshape(bm, D)
    o_ref[...] = yp.astype(jnp.bfloat16)


def _tc_matmul(x, w):
    return pl.pallas_call(
        _mm_body,
        grid=(NNP // _BM,),
        in_specs=[pl.BlockSpec((_BM, D), lambda i: (i, 0)),
                  pl.BlockSpec((D, D), lambda i: (0, 0))],
        out_specs=pl.BlockSpec((_BM, D), lambda i: (i, 0)),
        out_shape=jax.ShapeDtypeStruct((NNP, D), jnp.bfloat16),
    )(x, w)


def kernel(adj_indices, adj_values, pos_src, pos_dst, neg_src, neg_dst,
           emb_node, emb_attri, W1, b1, W2, b2):
    x0 = jnp.concatenate(
        [emb_node, emb_attri,
         jnp.zeros((NNP - NN, D), jnp.float32)], axis=0)
    row = adj_indices[0].astype(jnp.int32).reshape(EBLK, EK)
    col = adj_indices[1].astype(jnp.int32).reshape(EBLK, EK)
    vals = adj_values.astype(jnp.float32).reshape(EBLK, EK)
    idx_all = jnp.concatenate([pos_src, pos_dst, neg_src, neg_dst]).astype(jnp.int32)
    b1r = b1.reshape(1, D).astype(jnp.float32)
    b2r = b2.reshape(1, D).astype(jnp.float32)

    def as_i32_halves(s):
        return lax.bitcast_convert_type(
            s.reshape(NNP, D // 2, 2), jnp.int32).reshape(2 * NNP, DH // 2)

    g0 = _sc_gather(x0, idx_all)
    s1 = _tc_matmul(x0, W1)
    x1 = _sc_spmm(as_i32_halves(s1), row, col, vals, b1r)
    s2 = _tc_matmul(x1, W2)
    g1 = _sc_gather(x1, idx_all)
    x2 = _sc_spmm(as_i32_halves(s2), row, col, vals, b2r)
    g2 = _sc_gather(x2, idx_all)

    g0 = g0.reshape(4, B, D)
    g1 = g1.reshape(4, B, D)
    g2 = g2.reshape(4, B, D)
    src_emb = jnp.stack([g0[0], g1[0], g2[0]])
    dst_emb = jnp.stack([g0[1], g1[1], g2[1]])
    src_neg = jnp.stack([g0[2], g1[2], g2[2]])
    dst_neg = jnp.stack([g0[3], g1[3], g2[3]])
    return (src_emb, dst_emb, src_neg, dst_neg)


# speedup vs baseline: 1.5841x; 1.5841x over previous
import functools

import jax
import jax.numpy as jnp
from jax import lax
from jax.experimental import pallas as pl
from jax.experimental.pallas import tpu as pltpu
from jax.experimental.pallas import tpu_sc as plsc

N_NODE = 10000
N_ATTRI = 2000
NN = N_NODE + N_ATTRI
NNP = 12032
E = 384000
D = 128
DH = D // 2
B = 16384
GB = 4 * B

NC = 2
NS = 16
NW = NC * NS

EPT = E // NS
EK = 96
SUBS = 5
NBODY = EPT // (EK * SUBS)
EBLK = E // EK

RPT = NNP // NS
GPT = GB // NW
GK = 128
NGCH = GPT // GK

_mesh = plsc.VectorSubcoreMesh(core_axis_name="c", subcore_axis_name="s")


@functools.partial(
    pl.kernel,
    mesh=_mesh,
    compiler_params=pltpu.CompilerParams(use_tc_tiling_on_sc=False),
    out_type=jax.ShapeDtypeStruct((GB, D), jnp.float32),
    scratch_types=[
        pltpu.VMEM((GPT,), jnp.int32),
        pltpu.VMEM((GK, D), jnp.float32),
        pltpu.VMEM((GK, D), jnp.float32),
        pltpu.VMEM((GK, D), jnp.float32),
        pltpu.VMEM((GK, D), jnp.float32),
        pltpu.SemaphoreType.DMA,
        pltpu.SemaphoreType.DMA,
        pltpu.SemaphoreType.DMA,
    ],
)
def _sc_gather(x_hbm, idx_hbm, out_hbm, idxv, A0, A1, O0, O1,
               gsem, osem, isem):
    wid = lax.axis_index("s") * NC + lax.axis_index("c")
    base = wid * GPT
    A = (A0, A1)
    O = (O0, O1)

    pltpu.async_copy(idx_hbm.at[pl.ds(base, GPT)], idxv, isem).wait()
    for k in range(2):
        pltpu.async_copy(x_hbm.at[idxv.at[pl.ds(k * GK, GK)]], A[k], gsem)

    def norm_chunk(a, o):
        def grp(it, carry):
            for u in range(4):
                r = it * 4 + u
                av = [a[r, pl.ds(cc * 16, 16)] for cc in range(D // 16)]
                sq = [x * x for x in av]
                while len(sq) > 1:
                    sq = [sq[2 * t] + sq[2 * t + 1] for t in range(len(sq) // 2)]
                bc = [jnp.full((16,), sq[0][e], jnp.float32) for e in range(16)]
                while len(bc) > 1:
                    bc = [bc[2 * t] + bc[2 * t + 1] for t in range(len(bc) // 2)]
                ss = jnp.maximum(bc[0], 1e-24)
                iy = jnp.int32(0x5F3759DF) - lax.shift_right_logical(
                    lax.bitcast_convert_type(ss, jnp.int32), 1)
                y = lax.bitcast_convert_type(iy, jnp.float32)
                y = y * (1.5 - 0.5 * ss * y * y)
                y = y * (1.5 - 0.5 * ss * y * y)
                y = y * (1.5 - 0.5 * ss * y * y)
                for cc in range(D // 16):
                    o[r, pl.ds(cc * 16, 16)] = av[cc] * y
            return carry

        lax.fori_loop(0, GK // 4, grp, 0)

    def step(i, k):
        j = 2 * i + k

        def drain_out():
            pltpu.make_async_copy(O[k], out_hbm.at[pl.ds(0, GK)], osem).wait()

        def fire_gather():
            pltpu.async_copy(x_hbm.at[idxv.at[pl.ds((j + 2) * GK, GK)]],
                             A[k], gsem)

        pltpu.make_async_copy(
            x_hbm.at[idxv.at[pl.ds(0, GK)]], A[k], gsem).wait()
        pl.when(i > 0)(drain_out)
        norm_chunk(A[k], O[k])
        pltpu.async_copy(O[k], out_hbm.at[pl.ds(base + j * GK, GK)], osem)
        pl.when(i < NGCH // 2 - 1)(fire_gather)

    def loop(i, carry):
        step(i, 0)
        step(i, 1)
        return carry

    lax.fori_loop(0, NGCH // 2, loop, 0)
    for k in range(2):
        pltpu.make_async_copy(O[k], out_hbm.at[pl.ds(0, GK)], osem).wait()


@functools.partial(
    pl.kernel,
    mesh=_mesh,
    compiler_params=pltpu.CompilerParams(use_tc_tiling_on_sc=False),
    out_type=jax.ShapeDtypeStruct((NNP, D), jnp.float32),
    scratch_types=[
        pltpu.VMEM((SUBS, EK), jnp.int32),
        pltpu.VMEM((SUBS, EK), jnp.int32),
        pltpu.VMEM((SUBS, EK), jnp.int32),
        pltpu.VMEM((SUBS, EK), jnp.int32),
        pltpu.VMEM((SUBS, EK), jnp.float32),
        pltpu.VMEM((SUBS, EK), jnp.float32),
        *[pltpu.VMEM((EK, DH // 2), jnp.int32) for _ in range(SUBS)],
        *[pltpu.VMEM((EK, DH), jnp.float32) for _ in range(SUBS)],
        *[pltpu.VMEM((EK,), jnp.int32) for _ in range(SUBS)],
        pltpu.VMEM((1, DH), jnp.float32),
        pltpu.VMEM_SHARED((NNP, DH), jnp.float32),
        pltpu.SemaphoreType.DMA,
        pltpu.SemaphoreType.DMA,
        pltpu.SemaphoreType.DMA,
        pltpu.SemaphoreType.DMA,
    ],
)
def _sc_spmm(sup_hbm, row_hbm, col_hbm, val_hbm, b_hbm, out_hbm,
             row0, row1, col0, col1, val0, val1,
             G0, G1, G2, G3, G4, S0, S1, S2, S3, S4, R0, R1, R2, R3, R4,
             bv, acc, gsem, ssem, isem0, isem1):
    G = (G0, G1, G2, G3, G4)
    S = (S0, S1, S2, S3, S4)
    R = (R0, R1, R2, R3, R4)
    core = lax.axis_index("c")
    s = lax.axis_index("s")
    rb0 = s * (EPT // EK)

    isems = (isem0, isem1)
    rows = (row0, row1)
    cols = (col0, col1)
    vals = (val0, val1)

    def fire_idx(p, rb):
        pltpu.async_copy(row_hbm.at[pl.ds(rb, SUBS)], rows[p], isems[p])
        pltpu.async_copy(col_hbm.at[pl.ds(rb, SUBS)], cols[p], isems[p])
        pltpu.async_copy(val_hbm.at[pl.ds(rb, SUBS)], vals[p], isems[p])

    def wait_idx(p):
        pltpu.make_async_copy(row_hbm.at[pl.ds(0, SUBS)], rows[p], isems[p]).wait()
        pltpu.make_async_copy(col_hbm.at[pl.ds(0, SUBS)], cols[p], isems[p]).wait()
        pltpu.make_async_copy(val_hbm.at[pl.ds(0, SUBS)], vals[p], isems[p]).wait()

    def xform_col_and_fire_gather(p, k):
        for g in range(EK // 16):
            sl = pl.ds(g * 16, 16)
            cols[p][k, sl] = cols[p][k, sl] * 2 + core
        pltpu.async_copy(sup_hbm.at[cols[p].at[k]], G[k], gsem)

    fire_idx(0, rb0)
    fire_idx(1, rb0 + SUBS)

    pltpu.sync_copy(b_hbm.at[:, pl.ds(core * DH, DH)], bv)
    bias = [bv[0, pl.ds(cc * 16, 16)] for cc in range(DH // 16)]

    def zrow(r, carry):
        for cc in range(DH // 16):
            S0[r, pl.ds(cc * 16, 16)] = bias[cc]
        return carry

    lax.fori_loop(0, EK, zrow, 0)
    _echunks = [(j * EK, EK) for j in range(RPT // EK)] + [
        ((RPT // EK) * EK, RPT - (RPT // EK) * EK)]
    for off, ln in _echunks:
        pltpu.sync_copy(S0.at[pl.ds(0, ln)], acc.at[pl.ds(s * RPT + off, ln)])
    plsc.subcore_barrier()

    wait_idx(0)
    for k in range(SUBS):
        xform_col_and_fire_gather(0, k)

    def scale(p, k):
        def sgrp(g, carry):
            vv = vals[p][k, pl.ds(g * 16, 16)]
            base = g * 16
            for e in range(16):
                v = jnp.full((16,), vv[e], jnp.float32)
                r = base + e
                tmp = []
                for g2 in range(DH // 32):
                    w = G[k][r, pl.ds(g2 * 16, 16)]
                    lo = lax.bitcast_convert_type(
                        lax.shift_left(w, 16), jnp.float32)
                    hi = lax.bitcast_convert_type(
                        lax.bitwise_and(w, jnp.int32(-65536)), jnp.float32)
                    tmp += [lo, hi]
                for cc in range(DH // 16):
                    S[k][r, pl.ds(cc * 16, 16)] = tmp[cc] * v
            return carry

        lax.fori_loop(0, EK // 16, sgrp, 0)

    def body(i, p):
        n = 1 - p
        for k in range(SUBS):
            def drain():
                pltpu.make_async_copy(S[k], acc.at[R[k]], ssem).wait()

            if p == 0:
                pl.when(i > 0)(drain)
            else:
                drain()
            pltpu.make_async_copy(sup_hbm.at[cols[p].at[k]], G[k], gsem).wait()
            if k == 0:
                if p == 0:
                    wait_idx(n)
                else:
                    pl.when(i < NBODY // 2 - 1)(lambda: wait_idx(n))
            for g in range(EK // 16):
                sl = pl.ds(g * 16, 16)
                R[k][sl] = rows[p][k, sl]
            scale(p, k)
            pltpu.async_copy(S[k], acc.at[R[k]], ssem, add=True)
            if p == 0:
                xform_col_and_fire_gather(n, k)
            else:
                pl.when(i < NBODY // 2 - 1)(
                    functools.partial(xform_col_and_fire_gather, n, k))
            if k == SUBS - 1:
                pl.when(i < NBODY // 2 - 1)(
                    lambda: fire_idx(p, rb0 + (2 * i + 2 + p) * SUBS))

    def loop(i, carry):
        body(i, 0)
        body(i, 1)
        return carry

    lax.fori_loop(0, NBODY // 2, loop, 0)

    for k in range(SUBS):
        pltpu.make_async_copy(S[k], acc.at[R[k]], ssem).wait()
    plsc.subcore_barrier()
    for j, (off, ln) in enumerate(_echunks):
        gb = S[j % 2]
        pltpu.sync_copy(acc.at[pl.ds(s * RPT + off, ln)], gb.at[pl.ds(0, ln)])
        pltpu.sync_copy(gb.at[pl.ds(0, ln)],
                        out_hbm.at[pl.ds(s * RPT + off, ln),
                                   pl.ds(core * DH, DH)])


_BM = 752


def _rne16(v):
    b = lax.bitcast_convert_type(v, jnp.int32)
    r = lax.shift_right_logical(
        b + jnp.int32(0x7FFF)
        + lax.bitwise_and(lax.shift_right_logical(b, 16), jnp.int32(1)), 16)
    return lax.bitwise_and(r, jnp.int32(0xFFFF))


def _mm_body(x_ref, w_ref, o_ref):
    y = jnp.dot(x_ref[...], w_ref[...], preferred_element_type=jnp.float32)
    lo = jnp.concatenate([y[:, 32 * g:32 * g + 16] for g in range(4)], axis=1)
    hi = jnp.concatenate([y[:, 32 * g + 16:32 * g + 32] for g in range(4)],
                         axis=1)
    o_ref[...] = lax.bitwise_or(lax.shift_left(_rne16(hi), 16), _rne16(lo))


def _tc_matmul(x, w):
    return pl.pallas_call(
        _mm_body,
        grid=(NNP // _BM,),
        in_specs=[pl.BlockSpec((_BM, D), lambda i: (i, 0)),
                  pl.BlockSpec((D, D), lambda i: (0, 0))],
        out_specs=pl.BlockSpec((_BM, D // 2), lambda i: (i, 0)),
        out_shape=jax.ShapeDtypeStruct((NNP, D // 2), jnp.int32),
    )(x, w)


def kernel(adj_indices, adj_values, pos_src, pos_dst, neg_src, neg_dst,
           emb_node, emb_attri, W1, b1, W2, b2):
    x0 = jnp.concatenate(
        [emb_node, emb_attri,
         jnp.zeros((NNP - NN, D), jnp.float32)], axis=0)
    row = adj_indices[0].astype(jnp.int32).reshape(EBLK, EK)
    col = adj_indices[1].astype(jnp.int32).reshape(EBLK, EK)
    vals = adj_values.astype(jnp.float32).reshape(EBLK, EK)
    idx_all = jnp.concatenate([pos_src, pos_dst, neg_src, neg_dst]).astype(jnp.int32)
    b1r = b1.reshape(1, D).astype(jnp.float32)
    b2r = b2.reshape(1, D).astype(jnp.float32)

    def as_i32_halves(s):
        return s.reshape(2 * NNP, DH // 2)

    g0 = _sc_gather(x0, idx_all)
    s1 = _tc_matmul(x0, W1)
    x1 = _sc_spmm(as_i32_halves(s1), row, col, vals, b1r)
    s2 = _tc_matmul(x1, W2)
    g1 = _sc_gather(x1, idx_all)
    x2 = _sc_spmm(as_i32_halves(s2), row, col, vals, b2r)
    g2 = _sc_gather(x2, idx_all)

    g0 = g0.reshape(4, B, D)
    g1 = g1.reshape(4, B, D)
    g2 = g2.reshape(4, B, D)
    src_emb = jnp.stack([g0[0], g1[0], g2[0]])
    dst_emb = jnp.stack([g0[1], g1[1], g2[1]])
    src_neg = jnp.stack([g0[2], g1[2], g2[2]])
    dst_neg = jnp.stack([g0[3], g1[3], g2[3]])
    return (src_emb, dst_emb, src_neg, dst_neg)


# speedup vs baseline: 2.2973x; 1.4502x over previous
import functools

import jax
import jax.numpy as jnp
from jax import lax
from jax.experimental import pallas as pl
from jax.experimental.pallas import tpu as pltpu
from jax.experimental.pallas import tpu_sc as plsc

N_NODE = 10000
N_ATTRI = 2000
NN = N_NODE + N_ATTRI
NNP = 12032
E = 384000
D = 128
DH = D // 2
B = 16384
GB = 4 * B

NC = 2
NS = 16
NW = NC * NS

EPT = E // NS
EK = 96
SUBS = 5
NBODY = EPT // (EK * SUBS)
EBLK = E // EK

RPT = NNP // NS
GPT = GB // NW
GK = 128
NGCH = GPT // GK

_mesh = plsc.VectorSubcoreMesh(core_axis_name="c", subcore_axis_name="s")


@functools.partial(
    pl.kernel,
    mesh=_mesh,
    compiler_params=pltpu.CompilerParams(use_tc_tiling_on_sc=False),
    out_type=jax.ShapeDtypeStruct((GB, D), jnp.float32),
    scratch_types=[
        pltpu.VMEM((GPT,), jnp.int32),
        pltpu.VMEM((GK, D), jnp.float32),
        pltpu.VMEM((GK, D), jnp.float32),
        pltpu.VMEM((GK, D), jnp.float32),
        pltpu.VMEM((GK, D), jnp.float32),
        pltpu.SemaphoreType.DMA,
        pltpu.SemaphoreType.DMA,
        pltpu.SemaphoreType.DMA,
    ],
)
def _sc_gather(x_hbm, idx_hbm, out_hbm, idxv, A0, A1, O0, O1,
               gsem, osem, isem):
    wid = lax.axis_index("s") * NC + lax.axis_index("c")
    base = wid * GPT
    A = (A0, A1)
    O = (O0, O1)

    pltpu.async_copy(idx_hbm.at[pl.ds(base, GPT)], idxv, isem).wait()
    for k in range(2):
        pltpu.async_copy(x_hbm.at[idxv.at[pl.ds(k * GK, GK)]], A[k], gsem)

    def norm_chunk(a, o):
        def grp(it, carry):
            for u in range(4):
                r = it * 4 + u
                av = [a[r, pl.ds(cc * 16, 16)] for cc in range(D // 16)]
                sq = [x * x for x in av]
                while len(sq) > 1:
                    sq = [sq[2 * t] + sq[2 * t + 1] for t in range(len(sq) // 2)]
                bc = [jnp.full((16,), sq[0][e], jnp.float32) for e in range(16)]
                while len(bc) > 1:
                    bc = [bc[2 * t] + bc[2 * t + 1] for t in range(len(bc) // 2)]
                ss = jnp.maximum(bc[0], 1e-24)
                iy = jnp.int32(0x5F3759DF) - lax.shift_right_logical(
                    lax.bitcast_convert_type(ss, jnp.int32), 1)
                y = lax.bitcast_convert_type(iy, jnp.float32)
                y = y * (1.5 - 0.5 * ss * y * y)
                y = y * (1.5 - 0.5 * ss * y * y)
                y = y * (1.5 - 0.5 * ss * y * y)
                for cc in range(D // 16):
                    o[r, pl.ds(cc * 16, 16)] = av[cc] * y
            return carry

        lax.fori_loop(0, GK // 4, grp, 0)

    def step(i, k):
        j = 2 * i + k

        def drain_out():
            pltpu.make_async_copy(O[k], out_hbm.at[pl.ds(0, GK)], osem).wait()

        def fire_gather():
            pltpu.async_copy(x_hbm.at[idxv.at[pl.ds((j + 2) * GK, GK)]],
                             A[k], gsem)

        pltpu.make_async_copy(
            x_hbm.at[idxv.at[pl.ds(0, GK)]], A[k], gsem).wait()
        pl.when(i > 0)(drain_out)
        norm_chunk(A[k], O[k])
        pltpu.async_copy(O[k], out_hbm.at[pl.ds(base + j * GK, GK)], osem)
        pl.when(i < NGCH // 2 - 1)(fire_gather)

    def loop(i, carry):
        step(i, 0)
        step(i, 1)
        return carry

    lax.fori_loop(0, NGCH // 2, loop, 0)
    for k in range(2):
        pltpu.make_async_copy(O[k], out_hbm.at[pl.ds(0, GK)], osem).wait()


@functools.partial(
    pl.kernel,
    mesh=_mesh,
    compiler_params=pltpu.CompilerParams(use_tc_tiling_on_sc=False),
    out_type=jax.ShapeDtypeStruct((NNP, D), jnp.float32),
    scratch_types=[
        pltpu.VMEM((SUBS, EK), jnp.int32),
        pltpu.VMEM((SUBS, EK), jnp.int32),
        pltpu.VMEM((SUBS, EK), jnp.int32),
        pltpu.VMEM((SUBS, EK), jnp.int32),
        pltpu.VMEM((SUBS, EK), jnp.float32),
        pltpu.VMEM((SUBS, EK), jnp.float32),
        *[pltpu.VMEM((EK, DH), jnp.float32) for _ in range(SUBS)],
        *[pltpu.VMEM((EK, DH), jnp.float32) for _ in range(SUBS)],
        *[pltpu.VMEM((EK,), jnp.int32) for _ in range(SUBS)],
        pltpu.VMEM((1, DH), jnp.float32),
        pltpu.VMEM_SHARED((NNP, DH), jnp.float32),
        pltpu.SemaphoreType.DMA,
        pltpu.SemaphoreType.DMA,
        pltpu.SemaphoreType.DMA,
        pltpu.SemaphoreType.DMA,
    ],
)
def _sc_spmm(sup_hbm, row_hbm, col_hbm, val_hbm, b_hbm, out_hbm,
             row0, row1, col0, col1, val0, val1,
             G0, G1, G2, G3, G4, S0, S1, S2, S3, S4, R0, R1, R2, R3, R4,
             bv, acc, gsem, ssem, isem0, isem1):
    G = (G0, G1, G2, G3, G4)
    S = (S0, S1, S2, S3, S4)
    R = (R0, R1, R2, R3, R4)
    core = lax.axis_index("c")
    s = lax.axis_index("s")
    rb0 = s * (EPT // EK)

    isems = (isem0, isem1)
    rows = (row0, row1)
    cols = (col0, col1)
    vals = (val0, val1)

    def fire_idx(p, rb):
        pltpu.async_copy(row_hbm.at[pl.ds(rb, SUBS)], rows[p], isems[p])
        pltpu.async_copy(col_hbm.at[pl.ds(rb, SUBS)], cols[p], isems[p])
        pltpu.async_copy(val_hbm.at[pl.ds(rb, SUBS)], vals[p], isems[p])

    def wait_idx(p):
        pltpu.make_async_copy(row_hbm.at[pl.ds(0, SUBS)], rows[p], isems[p]).wait()
        pltpu.make_async_copy(col_hbm.at[pl.ds(0, SUBS)], cols[p], isems[p]).wait()
        pltpu.make_async_copy(val_hbm.at[pl.ds(0, SUBS)], vals[p], isems[p]).wait()

    def xform_col_and_fire_gather(p, k):
        for g in range(EK // 16):
            sl = pl.ds(g * 16, 16)
            cols[p][k, sl] = cols[p][k, sl] * 2 + core
        pltpu.async_copy(sup_hbm.at[cols[p].at[k]], G[k], gsem)

    fire_idx(0, rb0)
    fire_idx(1, rb0 + SUBS)

    pltpu.sync_copy(b_hbm.at[:, pl.ds(core * DH, DH)], bv)
    bias = [bv[0, pl.ds(cc * 16, 16)] for cc in range(DH // 16)]

    def zrow(r, carry):
        for cc in range(DH // 16):
            S0[r, pl.ds(cc * 16, 16)] = bias[cc]
        return carry

    lax.fori_loop(0, EK, zrow, 0)
    _echunks = [(j * EK, EK) for j in range(RPT // EK)] + [
        ((RPT // EK) * EK, RPT - (RPT // EK) * EK)]
    for off, ln in _echunks:
        pltpu.sync_copy(S0.at[pl.ds(0, ln)], acc.at[pl.ds(s * RPT + off, ln)])
    plsc.subcore_barrier()

    wait_idx(0)
    for k in range(SUBS):
        xform_col_and_fire_gather(0, k)

    def scale(p, k):
        def sgrp(g, carry):
            vv = vals[p][k, pl.ds(g * 16, 16)]
            base = g * 16
            for e in range(16):
                v = jnp.full((16,), vv[e], jnp.float32)
                r = base + e
                tmp = [G[k][r, pl.ds(cc * 16, 16)] for cc in range(DH // 16)]
                for cc in range(DH // 16):
                    S[k][r, pl.ds(cc * 16, 16)] = tmp[cc] * v
            return carry

        lax.fori_loop(0, EK // 16, sgrp, 0)

    def body(i, p):
        n = 1 - p
        for k in range(SUBS):
            def drain():
                pltpu.make_async_copy(S[k], acc.at[R[k]], ssem).wait()

            if p == 0:
                pl.when(i > 0)(drain)
            else:
                drain()
            pltpu.make_async_copy(sup_hbm.at[cols[p].at[k]], G[k], gsem).wait()
            if k == 0:
                if p == 0:
                    wait_idx(n)
                else:
                    pl.when(i < NBODY // 2 - 1)(lambda: wait_idx(n))
            for g in range(EK // 16):
                sl = pl.ds(g * 16, 16)
                R[k][sl] = rows[p][k, sl]
            scale(p, k)
            pltpu.async_copy(S[k], acc.at[R[k]], ssem, add=True)
            if p == 0:
                xform_col_and_fire_gather(n, k)
            else:
                pl.when(i < NBODY // 2 - 1)(
                    functools.partial(xform_col_and_fire_gather, n, k))
            if k == SUBS - 1:
                pl.when(i < NBODY // 2 - 1)(
                    lambda: fire_idx(p, rb0 + (2 * i + 2 + p) * SUBS))

    def loop(i, carry):
        body(i, 0)
        body(i, 1)
        return carry

    lax.fori_loop(0, NBODY // 2, loop, 0)

    for k in range(SUBS):
        pltpu.make_async_copy(S[k], acc.at[R[k]], ssem).wait()
    plsc.subcore_barrier()
    for j, (off, ln) in enumerate(_echunks):
        gb = S[j % 2]
        pltpu.sync_copy(acc.at[pl.ds(s * RPT + off, ln)], gb.at[pl.ds(0, ln)])
        pltpu.sync_copy(gb.at[pl.ds(0, ln)],
                        out_hbm.at[pl.ds(s * RPT + off, ln),
                                   pl.ds(core * DH, DH)])


_BM = 752


def _mm_body(x_ref, w_ref, o_ref):
    o_ref[...] = jnp.dot(x_ref[...], w_ref[...],
                         preferred_element_type=jnp.float32)


def _tc_matmul(x, w):
    return pl.pallas_call(
        _mm_body,
        grid=(NNP // _BM,),
        in_specs=[pl.BlockSpec((_BM, D), lambda i: (i, 0)),
                  pl.BlockSpec((D, D), lambda i: (0, 0))],
        out_specs=pl.BlockSpec((_BM, D), lambda i: (i, 0)),
        out_shape=jax.ShapeDtypeStruct((NNP, D), jnp.float32),
    )(x, w)


def kernel(adj_indices, adj_values, pos_src, pos_dst, neg_src, neg_dst,
           emb_node, emb_attri, W1, b1, W2, b2):
    x0 = jnp.concatenate(
        [emb_node, emb_attri,
         jnp.zeros((NNP - NN, D), jnp.float32)], axis=0)
    row = adj_indices[0].astype(jnp.int32).reshape(EBLK, EK)
    col = adj_indices[1].astype(jnp.int32).reshape(EBLK, EK)
    vals = adj_values.astype(jnp.float32).reshape(EBLK, EK)
    idx_all = jnp.concatenate([pos_src, pos_dst, neg_src, neg_dst]).astype(jnp.int32)
    b1r = b1.reshape(1, D).astype(jnp.float32)
    b2r = b2.reshape(1, D).astype(jnp.float32)

    g0 = _sc_gather(x0, idx_all)
    s1 = _tc_matmul(x0, W1)
    x1 = _sc_spmm(s1.reshape(2 * NNP, DH), row, col, vals, b1r)
    s2 = _tc_matmul(x1, W2)
    g1 = _sc_gather(x1, idx_all)
    x2 = _sc_spmm(s2.reshape(2 * NNP, DH), row, col, vals, b2r)
    g2 = _sc_gather(x2, idx_all)

    g0 = g0.reshape(4, B, D)
    g1 = g1.reshape(4, B, D)
    g2 = g2.reshape(4, B, D)
    src_emb = jnp.stack([g0[0], g1[0], g2[0]])
    dst_emb = jnp.stack([g0[1], g1[1], g2[1]])
    src_neg = jnp.stack([g0[2], g1[2], g2[2]])
    dst_neg = jnp.stack([g0[3], g1[3], g2[3]])
    return (src_emb, dst_emb, src_neg, dst_neg)
